# baseline (speedup 1.0000x reference)
import dataclasses
import functools

import jax
import jax.numpy as jnp
from jax import lax
from jax.experimental import pallas as pl
from jax.experimental.pallas import tpu as pltpu
from jax.experimental.pallas import tpu_sc as plsc

_NC = 2
_NS = 16
_NW = _NC * _NS
_L = 16

_LN2 = 0.6931471805599453
_SQRT2 = 1.4142135623730951


def _ln16(x):
    bits = lax.bitcast_convert_type(x, jnp.int32)
    e = (bits >> 23) - 127
    m = lax.bitcast_convert_type(
        (bits & jnp.int32(0x007FFFFF)) | jnp.int32(0x3F800000), jnp.float32)
    big = m > _SQRT2
    m = jnp.where(big, m * 0.5, m)
    e = jnp.where(big, e + 1, e)
    t = m - 1.0
    z = t * t
    p = jnp.float32(7.0376836292e-2)
    p = p * t + jnp.float32(-1.1514610310e-1)
    p = p * t + jnp.float32(1.1676998740e-1)
    p = p * t + jnp.float32(-1.2420140846e-1)
    p = p * t + jnp.float32(1.4249322787e-1)
    p = p * t + jnp.float32(-1.6668057665e-1)
    p = p * t + jnp.float32(2.0000714765e-1)
    p = p * t + jnp.float32(-2.4999993993e-1)
    p = p * t + jnp.float32(3.3333331174e-1)
    y = t * z * p - 0.5 * z
    return t + y + e.astype(jnp.float32) * jnp.float32(_LN2)


@functools.partial(jax.jit, static_argnames=("n", "c", "p", "ch"))
def _sc_loss(pcl_flat, labels_p, w_p, pc_labels, pc_probs, pc_count, img_w,
             im_p, *, n, c, p, ch):
    n_slices = ch // _L
    p_per_w = p // _NW
    mesh = plsc.VectorSubcoreMesh(core_axis_name="c", subcore_axis_name="s")
    cp = pltpu.CompilerParams()
    if "needs_layout_passes" in pltpu.CompilerParams.__dataclass_fields__:
        cp = dataclasses.replace(cp, needs_layout_passes=False)

    @functools.partial(
        pl.kernel,
        out_type=jax.ShapeDtypeStruct((_NC, _L), jnp.float32),
        mesh=mesh,
        compiler_params=cp,
        scratch_types=[
            pltpu.VMEM((ch,), jnp.int32),
            pltpu.VMEM((ch,), jnp.float32),
            pltpu.VMEM((ch,), jnp.int32),
            pltpu.VMEM((ch,), jnp.float32),
            pltpu.VMEM((p_per_w,), jnp.int32),
            pltpu.VMEM((p_per_w,), jnp.float32),
            pltpu.VMEM((p_per_w,), jnp.float32),
            pltpu.VMEM((p_per_w,), jnp.float32),
            pltpu.VMEM((im_p.shape[0],), jnp.float32),
            pltpu.VMEM((_L,), jnp.float32),
            pltpu.VMEM((_NS * _L,), jnp.float32),
            pltpu.VMEM_SHARED((_NS * _L,), jnp.float32),
        ],
    )
    def k(pcl_ref, lab_ref, w_ref, pclab_ref, pcp_ref, pcc_ref, imw_ref,
          im_ref, out_ref, idx_v, p_v, lab_v, w_v, pclab_v, pcp_v, pcc_v,
          imw_v, im_v, res_v, all_v, stage):
        cid = lax.axis_index("c")
        sid = lax.axis_index("s")
        wid = cid * _NS + sid
        base = wid * ch
        lane = lax.iota(jnp.int32, _L)

        @pl.loop(0, n_slices)
        def _(s):
            row = base + s * _L + lane
            row = jnp.minimum(row, n - 1)
            idx_v[pl.ds(s * _L, _L)] = row * c

        pltpu.sync_copy(pcl_ref.at[idx_v], p_v)
        pltpu.sync_copy(lab_ref.at[pl.ds(base, ch)], lab_v)
        pltpu.sync_copy(w_ref.at[pl.ds(base, ch)], w_v)
        pb = wid * p_per_w
        pltpu.sync_copy(pclab_ref.at[pl.ds(pb, p_per_w)], pclab_v)
        pltpu.sync_copy(pcp_ref.at[pl.ds(pb, p_per_w)], pcp_v)
        pltpu.sync_copy(pcc_ref.at[pl.ds(pb, p_per_w)], pcc_v)
        pltpu.sync_copy(imw_ref.at[pl.ds(pb, p_per_w)], imw_v)
        pltpu.sync_copy(im_ref, im_v)

        def bg_body(s, acc):
            sl = pl.ds(s * _L, _L)
            contrib = jnp.where(lab_v[sl] == 0, w_v[sl] * _ln16(p_v[sl]), 0.0)
            return acc + contrib

        bg = lax.fori_loop(0, n_slices, bg_body, jnp.zeros((_L,), jnp.float32))

        zero_idx = jnp.zeros((_L,), jnp.int32)
        im0 = plsc.load_gather(im_v, [zero_idx])
        bg_act = jnp.where(im0 != 0.0, 1.0, 0.0)

        pclab = pclab_v[...]
        im_at = plsc.load_gather(
            im_v, [jnp.clip(pclab, 0, im_p.shape[0] - 1)])
        fg_mask = (im_at != 0.0) & (pclab > 0) & (pcc_v[...] > 0.0)
        fg = jnp.where(fg_mask, imw_v[...] * _ln16(pcp_v[...]), 0.0)

        res_v[...] = bg * bg_act + fg
        pltpu.sync_copy(res_v, stage.at[pl.ds(sid * _L, _L)])
        plsc.subcore_barrier()

        @pl.when(sid == 0)
        def _():
            pltpu.sync_copy(stage, all_v)

            def red_body(r, acc):
                return acc + all_v[pl.ds(r * _L, _L)]

            tot = lax.fori_loop(0, _NS, red_body,
                                jnp.zeros((_L,), jnp.float32))
            val = jnp.sum(tot) * jnp.float32(-1.0 / n)
            res_v[...] = jnp.full((_L,), val, jnp.float32)
            pltpu.sync_copy(res_v, out_ref.at[cid])

    return k(pcl_flat, labels_p, w_p, pc_labels, pc_probs, pc_count, img_w,
             im_p)


def kernel(pcl_prob, labels, cls_loss_weights, gt_assignment, pc_labels,
           pc_probs, pc_count, img_cls_loss_weights, im_labels_real):
    n, c = pcl_prob.shape
    p = pc_labels.shape[0]
    ch = -(-n // (_NW * _L)) * _L
    pad = _NW * ch - n
    labels_p = jnp.concatenate(
        [labels, jnp.ones((pad,), labels.dtype)]) if pad else labels
    w_p = jnp.concatenate(
        [cls_loss_weights, jnp.zeros((pad,), cls_loss_weights.dtype)]
    ) if pad else cls_loss_weights
    im_pad = -(-c // 8) * 8 - c
    im_p = jnp.concatenate(
        [im_labels_real, jnp.zeros((im_pad,), im_labels_real.dtype)]
    ) if im_pad else im_labels_real
    out = _sc_loss(pcl_prob.reshape(-1), labels_p, w_p, pc_labels, pc_probs,
                   pc_count, img_cls_loss_weights, im_p,
                   n=n, c=c, p=p, ch=ch)
    return out[0, 0] + out[1, 0]


# speedup vs baseline: 1.0603x; 1.0603x over previous
import dataclasses
import functools

import jax
import jax.numpy as jnp
from jax import lax
from jax.experimental import pallas as pl
from jax.experimental.pallas import tpu as pltpu
from jax.experimental.pallas import tpu_sc as plsc

_NC = 2
_NS = 16
_NW = _NC * _NS
_L = 16

_LN2 = 0.6931471805599453
_SQRT2 = 1.4142135623730951


def _ln16(x):
    bits = lax.bitcast_convert_type(x, jnp.int32)
    e = (bits >> 23) - 127
    m = lax.bitcast_convert_type(
        (bits & jnp.int32(0x007FFFFF)) | jnp.int32(0x3F800000), jnp.float32)
    big = m > _SQRT2
    m = jnp.where(big, m * 0.5, m)
    e = jnp.where(big, e + 1, e)
    t = m - 1.0
    z = t * t
    p = jnp.float32(7.0376836292e-2)
    p = p * t + jnp.float32(-1.1514610310e-1)
    p = p * t + jnp.float32(1.1676998740e-1)
    p = p * t + jnp.float32(-1.2420140846e-1)
    p = p * t + jnp.float32(1.4249322787e-1)
    p = p * t + jnp.float32(-1.6668057665e-1)
    p = p * t + jnp.float32(2.0000714765e-1)
    p = p * t + jnp.float32(-2.4999993993e-1)
    p = p * t + jnp.float32(3.3333331174e-1)
    y = t * z * p - 0.5 * z
    return t + y + e.astype(jnp.float32) * jnp.float32(_LN2)


@functools.partial(jax.jit, static_argnames=("n", "c", "p", "ch"))
def _sc_loss(pcl_flat, labels, w, pc_labels, pc_probs, pc_count, img_w,
             im_labels, *, n, c, p, ch):
    n_slices = ch // _L
    p_per_w = p // _NW
    mesh = plsc.VectorSubcoreMesh(core_axis_name="c", subcore_axis_name="s")
    cp = pltpu.CompilerParams()
    if "needs_layout_passes" in pltpu.CompilerParams.__dataclass_fields__:
        cp = dataclasses.replace(cp, needs_layout_passes=False)

    @functools.partial(
        pl.kernel,
        out_type=jax.ShapeDtypeStruct((_NC, _L), jnp.float32),
        mesh=mesh,
        compiler_params=cp,
        scratch_types=[
            pltpu.VMEM((ch,), jnp.int32),
            pltpu.VMEM((ch,), jnp.float32),
            pltpu.VMEM((ch,), jnp.int32),
            pltpu.VMEM((ch,), jnp.float32),
            pltpu.VMEM((p_per_w,), jnp.int32),
            pltpu.VMEM((p_per_w,), jnp.float32),
            pltpu.VMEM((p_per_w,), jnp.float32),
            pltpu.VMEM((p_per_w,), jnp.float32),
            pltpu.VMEM((c,), jnp.float32),
            pltpu.VMEM((_L,), jnp.float32),
            pltpu.VMEM((_NS * _L,), jnp.float32),
            pltpu.VMEM_SHARED((_NS * _L,), jnp.float32),
            pltpu.SemaphoreType.DMA,
            pltpu.SemaphoreType.DMA,
        ],
    )
    def k(pcl_ref, lab_ref, w_ref, pclab_ref, pcp_ref, pcc_ref, imw_ref,
          im_ref, out_ref, idx_v, p_v, lab_v, w_v, pclab_v, pcp_v, pcc_v,
          imw_v, im_v, res_v, all_v, stage, sem_bg, sem_fg):
        cid = lax.axis_index("c")
        sid = lax.axis_index("s")
        wid = cid * _NS + sid
        base = jnp.minimum(wid * ch, n - ch)
        start_off = wid * ch - base
        lane = lax.iota(jnp.int32, _L)

        pb = wid * p_per_w
        fg_cp = [
            pltpu.async_copy(pclab_ref.at[pl.ds(pb, p_per_w)], pclab_v, sem_fg),
            pltpu.async_copy(pcp_ref.at[pl.ds(pb, p_per_w)], pcp_v, sem_fg),
            pltpu.async_copy(pcc_ref.at[pl.ds(pb, p_per_w)], pcc_v, sem_fg),
            pltpu.async_copy(imw_ref.at[pl.ds(pb, p_per_w)], imw_v, sem_fg),
            pltpu.async_copy(im_ref, im_v, sem_fg),
        ]
        bg_cp = [
            pltpu.async_copy(lab_ref.at[pl.ds(base, ch)], lab_v, sem_bg),
            pltpu.async_copy(w_ref.at[pl.ds(base, ch)], w_v, sem_bg),
        ]

        @pl.loop(0, n_slices)
        def _(s):
            idx_v[pl.ds(s * _L, _L)] = (base + s * _L + lane) * c

        bg_cp.append(pltpu.async_copy(pcl_ref.at[idx_v], p_v, sem_bg))

        for h in fg_cp:
            h.wait()
        pclab = pclab_v[...]
        im_at = plsc.load_gather(im_v, [jnp.clip(pclab, 0, c - 1)])
        fg_mask = (im_at != 0.0) & (pclab > 0) & (pcc_v[...] > 0.0)
        fg = jnp.where(fg_mask, imw_v[...] * _ln16(pcp_v[...]), 0.0)

        im0 = plsc.load_gather(im_v, [jnp.zeros((_L,), jnp.int32)])
        bg_act = jnp.where(im0 != 0.0, 1.0, 0.0)

        for h in bg_cp:
            h.wait()

        def bg_body(s, acc):
            sl = pl.ds(s * _L, _L)
            off = s * _L + lane
            m = (off >= start_off) & (lab_v[sl] == 0)
            return acc + jnp.where(m, w_v[sl] * _ln16(p_v[sl]), 0.0)

        bg = lax.fori_loop(0, n_slices, bg_body, jnp.zeros((_L,), jnp.float32))

        res_v[...] = bg * bg_act + fg
        pltpu.sync_copy(res_v, stage.at[pl.ds(sid * _L, _L)])
        plsc.subcore_barrier()

        @pl.when(sid == 0)
        def _():
            pltpu.sync_copy(stage, all_v)

            def red_body(r, acc):
                return acc + all_v[pl.ds(r * _L, _L)]

            tot = lax.fori_loop(0, _NS, red_body,
                                jnp.zeros((_L,), jnp.float32))
            val = jnp.sum(tot) * jnp.float32(-1.0 / n)
            res_v[...] = jnp.full((_L,), val, jnp.float32)
            pltpu.sync_copy(res_v, out_ref.at[cid])

    return k(pcl_flat, labels, w, pc_labels, pc_probs, pc_count, img_w,
             im_labels)


def kernel(pcl_prob, labels, cls_loss_weights, gt_assignment, pc_labels,
           pc_probs, pc_count, img_cls_loss_weights, im_labels_real):
    n, c = pcl_prob.shape
    p = pc_labels.shape[0]
    ch = -(-n // (_NW * _L)) * _L
    out = _sc_loss(pcl_prob.reshape(-1), labels, cls_loss_weights, pc_labels,
                   pc_probs, pc_count, img_cls_loss_weights, im_labels_real,
                   n=n, c=c, p=p, ch=ch)
    return out[0, 0] + out[1, 0]


# speedup vs baseline: 1.5820x; 1.4920x over previous
import dataclasses
import functools

import jax
import jax.numpy as jnp
from jax import lax
from jax.experimental import pallas as pl
from jax.experimental.pallas import tpu as pltpu
from jax.experimental.pallas import tpu_sc as plsc

_NC = 2
_NS = 16
_NW = _NC * _NS
_L = 16

_LN2 = 0.6931471805599453
_SQRT2 = 1.4142135623730951


def _ln16(x):
    bits = lax.bitcast_convert_type(x, jnp.int32)
    e = (bits >> 23) - 127
    m = lax.bitcast_convert_type(
        (bits & jnp.int32(0x007FFFFF)) | jnp.int32(0x3F800000), jnp.float32)
    big = m > _SQRT2
    m = jnp.where(big, m * 0.5, m)
    e = jnp.where(big, e + 1, e)
    t = m - 1.0
    z = t * t
    p = jnp.float32(7.0376836292e-2)
    p = p * t + jnp.float32(-1.1514610310e-1)
    p = p * t + jnp.float32(1.1676998740e-1)
    p = p * t + jnp.float32(-1.2420140846e-1)
    p = p * t + jnp.float32(1.4249322787e-1)
    p = p * t + jnp.float32(-1.6668057665e-1)
    p = p * t + jnp.float32(2.0000714765e-1)
    p = p * t + jnp.float32(-2.4999993993e-1)
    p = p * t + jnp.float32(3.3333331174e-1)
    y = t * z * p - 0.5 * z
    return t + y + e.astype(jnp.float32) * jnp.float32(_LN2)


def _sc_fg(pc_labels, pc_probs, pc_count, img_w, im_labels, *, p, c):
    p_per_w = p // _NW
    mesh = plsc.VectorSubcoreMesh(core_axis_name="c", subcore_axis_name="s")
    cp = pltpu.CompilerParams()
    if "needs_layout_passes" in pltpu.CompilerParams.__dataclass_fields__:
        cp = dataclasses.replace(cp, needs_layout_passes=False)

    @functools.partial(
        pl.kernel,
        out_type=jax.ShapeDtypeStruct((_NC, _L), jnp.float32),
        mesh=mesh,
        compiler_params=cp,
        scratch_types=[
            pltpu.VMEM((p_per_w,), jnp.int32),
            pltpu.VMEM((p_per_w,), jnp.float32),
            pltpu.VMEM((p_per_w,), jnp.float32),
            pltpu.VMEM((p_per_w,), jnp.float32),
            pltpu.VMEM((c,), jnp.float32),
            pltpu.VMEM((_L,), jnp.float32),
            pltpu.VMEM((_NS * _L,), jnp.float32),
            pltpu.VMEM_SHARED((_NS * _L,), jnp.float32),
            pltpu.SemaphoreType.DMA,
        ],
    )
    def k(pclab_ref, pcp_ref, pcc_ref, imw_ref, im_ref, out_ref,
          pclab_v, pcp_v, pcc_v, imw_v, im_v, res_v, all_v, stage, sem):
        cid = lax.axis_index("c")
        sid = lax.axis_index("s")
        wid = cid * _NS + sid
        pb = wid * p_per_w
        cps = [
            pltpu.async_copy(pclab_ref.at[pl.ds(pb, p_per_w)], pclab_v, sem),
            pltpu.async_copy(pcp_ref.at[pl.ds(pb, p_per_w)], pcp_v, sem),
            pltpu.async_copy(pcc_ref.at[pl.ds(pb, p_per_w)], pcc_v, sem),
            pltpu.async_copy(imw_ref.at[pl.ds(pb, p_per_w)], imw_v, sem),
            pltpu.async_copy(im_ref, im_v, sem),
        ]
        for h in cps:
            h.wait()

        pclab = pclab_v[...]
        im_at = plsc.load_gather(im_v, [jnp.clip(pclab, 0, c - 1)])
        fg_mask = (im_at != 0.0) & (pclab > 0) & (pcc_v[...] > 0.0)
        fg = jnp.where(fg_mask, imw_v[...] * _ln16(pcp_v[...]), 0.0)

        res_v[...] = fg
        pltpu.sync_copy(res_v, stage.at[pl.ds(sid * _L, _L)])
        plsc.subcore_barrier()

        @pl.when(sid == 0)
        def _():
            pltpu.sync_copy(stage, all_v)

            def red_body(r, acc):
                return acc + all_v[pl.ds(r * _L, _L)]

            tot = lax.fori_loop(0, _NS, red_body,
                                jnp.zeros((_L,), jnp.float32))
            fg_sum = jnp.sum(tot)
            im0 = plsc.load_gather(im_v, [jnp.zeros((_L,), jnp.int32)])
            gate = jnp.where(im0 != 0.0, 1.0, 0.0)
            lane = lax.iota(jnp.int32, _L)
            res_v[...] = jnp.where(
                lane == 0, jnp.full((_L,), fg_sum, jnp.float32),
                jnp.where(lane == 1, gate, 0.0))
            pltpu.sync_copy(res_v, out_ref.at[cid])

    return k(pc_labels, pc_probs, pc_count, img_w, im_labels)


def _tc_bg_body(pcl_ref, lab_ref, w_ref, out_ref, *, n, bn):
    i = pl.program_id(0)
    base = i * bn
    valid2 = base + lax.broadcasted_iota(jnp.int32, (bn, 1), 0) < n
    z = jnp.log(jnp.where(valid2, pcl_ref[:, 0:1], 1.0))
    valid1 = base + lax.broadcasted_iota(jnp.int32, (bn,), 0) < n
    wm = jnp.where(valid1 & (lab_ref[...] == 0), w_ref[...], 0.0)
    partial = lax.dot_general(
        wm.reshape(1, -1), z,
        dimension_numbers=(((1,), (0,)), ((), ())),
        precision=lax.Precision.HIGHEST,
        preferred_element_type=jnp.float32)

    @pl.when(i == 0)
    def _():
        out_ref[...] = jnp.zeros_like(out_ref)

    out_ref[...] += partial


def _tc_bg(pcl_prob, labels, w, *, n, c, bn):
    grid = -(-n // bn)
    return pl.pallas_call(
        functools.partial(_tc_bg_body, n=n, bn=bn),
        grid=(grid,),
        in_specs=[
            pl.BlockSpec((bn, c), lambda i: (i, 0)),
            pl.BlockSpec((bn,), lambda i: (i,)),
            pl.BlockSpec((bn,), lambda i: (i,)),
        ],
        out_specs=pl.BlockSpec((1, 1), lambda i: (0, 0)),
        out_shape=jax.ShapeDtypeStruct((1, 1), jnp.float32),
    )(pcl_prob, labels, w)


@functools.partial(jax.jit, static_argnames=("n", "c", "p", "bn"))
def _loss(pcl_prob, labels, w, pc_labels, pc_probs, pc_count, img_w,
          im_labels, *, n, c, p, bn):
    fg_out = _sc_fg(pc_labels, pc_probs, pc_count, img_w, im_labels, p=p, c=c)
    bg = _tc_bg(pcl_prob, labels, w, n=n, c=c, bn=bn)
    gate = fg_out[0, 1]
    fg = fg_out[0, 0] + fg_out[1, 0]
    return -(bg[0, 0] * gate + fg) / n


def kernel(pcl_prob, labels, cls_loss_weights, gt_assignment, pc_labels,
           pc_probs, pc_count, img_cls_loss_weights, im_labels_real):
    n, c = pcl_prob.shape
    p = pc_labels.shape[0]
    bn = 1024
    return _loss(pcl_prob, labels, cls_loss_weights, pc_labels, pc_probs,
                 pc_count, img_cls_loss_weights, im_labels_real,
                 n=n, c=c, p=p, bn=bn)


# speedup vs baseline: 3.0265x; 1.9131x over previous
import functools

import jax
import jax.numpy as jnp
from jax import lax
from jax.experimental import pallas as pl


def _body(pcl_ref, lab_ref, w_ref, pclab_ref, pcp_ref, pcc_ref, imw_ref,
          im_ref, out_ref, *, n, c, p, bn):
    i = pl.program_id(0)
    base = i * bn
    im_r = im_ref[...].reshape(1, c)
    gate = (im_r[:, 0:1] != 0.0).astype(jnp.float32)

    valid2 = base + lax.broadcasted_iota(jnp.int32, (bn, 1), 0) < n
    z = jnp.log(jnp.where(valid2, pcl_ref[:, 0:1], 1.0))
    valid1 = base + lax.broadcasted_iota(jnp.int32, (bn,), 0) < n
    wm = jnp.where(valid1 & (lab_ref[...] == 0), w_ref[...], 0.0)
    bg_part = lax.dot_general(
        wm.reshape(1, bn), z,
        dimension_numbers=(((1,), (0,)), ((), ())),
        preferred_element_type=jnp.float32)

    @pl.when(i == 0)
    def _():
        pclab = pclab_ref[...].reshape(1, p)
        imnz = (im_r != 0.0).astype(jnp.float32)
        onehot = (lax.broadcasted_iota(jnp.int32, (c, p), 0)
                  == pclab).astype(jnp.float32)
        im_at_nz = lax.dot_general(
            imnz, onehot,
            dimension_numbers=(((1,), (0,)), ((), ())),
            preferred_element_type=jnp.float32)
        pcp = pcp_ref[...].reshape(1, p)
        fg_mask = ((im_at_nz > 0.5) & (pclab > 0)
                   & (pcc_ref[...].reshape(1, p) > 0.0))
        fg = jnp.sum(
            jnp.where(fg_mask,
                      imw_ref[...].reshape(1, p) * jnp.log(pcp), 0.0),
            keepdims=True)
        out_ref[...] = fg * jnp.float32(-1.0 / n)

    out_ref[...] += (gate * bg_part) * jnp.float32(-1.0 / n)


@functools.partial(jax.jit, static_argnames=("n", "c", "p", "bn"))
def _loss(pcl_prob, labels, w, pc_labels, pc_probs, pc_count, img_w,
          im_labels, *, n, c, p, bn):
    grid = -(-n // bn)
    full = lambda i: (0,)
    out = pl.pallas_call(
        functools.partial(_body, n=n, c=c, p=p, bn=bn),
        grid=(grid,),
        in_specs=[
            pl.BlockSpec((bn, c), lambda i: (i, 0)),
            pl.BlockSpec((bn,), lambda i: (i,)),
            pl.BlockSpec((bn,), lambda i: (i,)),
            pl.BlockSpec((p,), full),
            pl.BlockSpec((p,), full),
            pl.BlockSpec((p,), full),
            pl.BlockSpec((p,), full),
            pl.BlockSpec((c,), full),
        ],
        out_specs=pl.BlockSpec((1, 1), lambda i: (0, 0)),
        out_shape=jax.ShapeDtypeStruct((1, 1), jnp.float32),
    )(pcl_prob, labels, w, pc_labels, pc_probs, pc_count, img_w, im_labels)
    return out[0, 0]


def kernel(pcl_prob, labels, cls_loss_weights, gt_assignment, pc_labels,
           pc_probs, pc_count, img_cls_loss_weights, im_labels_real):
    n, c = pcl_prob.shape
    p = pc_labels.shape[0]
    return _loss(pcl_prob, labels, cls_loss_weights, pc_labels, pc_probs,
                 pc_count, img_cls_loss_weights, im_labels_real,
                 n=n, c=c, p=p, bn=1024)


# speedup vs baseline: 4.3967x; 1.4528x over previous
import functools

import jax
import jax.numpy as jnp
from jax import lax
from jax.experimental import pallas as pl


def _body(pcl_ref, lab_ref, w_ref, pclab_ref, pcp_ref, pcc_ref, imw_ref,
          im_ref, out_ref, *, n, c, p, bn):
    i = pl.program_id(0)
    base = i * bn
    im_r = im_ref[...].reshape(1, c)
    gate = (im_r[:, 0:1] != 0.0).astype(jnp.float32)

    x = pcl_ref[:, 0:1]
    z = jnp.log(jnp.where(x > 0.0, x, 1.0))
    valid1 = base + lax.broadcasted_iota(jnp.int32, (bn,), 0) < n
    wm = jnp.where(valid1 & (lab_ref[...] == 0), w_ref[...], 0.0)
    bg_part = lax.dot_general(
        wm.reshape(1, bn), z,
        dimension_numbers=(((1,), (0,)), ((), ())),
        preferred_element_type=jnp.float32)

    @pl.when(i == 0)
    def _():
        pclab = pclab_ref[...].reshape(1, p)
        imnz = (im_r != 0.0).astype(jnp.float32)
        onehot = (lax.broadcasted_iota(jnp.int32, (c, p), 0)
                  == pclab).astype(jnp.float32)
        im_at_nz = lax.dot_general(
            imnz, onehot,
            dimension_numbers=(((1,), (0,)), ((), ())),
            preferred_element_type=jnp.float32)
        pcp = pcp_ref[...].reshape(1, p)
        fg_mask = ((im_at_nz > 0.5) & (pclab > 0)
                   & (pcc_ref[...].reshape(1, p) > 0.0))
        fg = jnp.sum(
            jnp.where(fg_mask,
                      imw_ref[...].reshape(1, p) * jnp.log(pcp), 0.0),
            keepdims=True)
        out_ref[...] = fg * jnp.float32(-1.0 / n)

    out_ref[...] += (gate * bg_part) * jnp.float32(-1.0 / n)


@functools.partial(jax.jit, static_argnames=("n", "c", "p", "bn"))
def _loss(pcl_prob, labels, w, pc_labels, pc_probs, pc_count, img_w,
          im_labels, *, n, c, p, bn):
    grid = -(-n // bn)
    full = lambda i: (0,)
    out = pl.pallas_call(
        functools.partial(_body, n=n, c=c, p=p, bn=bn),
        grid=(grid,),
        in_specs=[
            pl.BlockSpec((bn, c), lambda i: (i, 0)),
            pl.BlockSpec((bn,), lambda i: (i,)),
            pl.BlockSpec((bn,), lambda i: (i,)),
            pl.BlockSpec((p,), full),
            pl.BlockSpec((p,), full),
            pl.BlockSpec((p,), full),
            pl.BlockSpec((p,), full),
            pl.BlockSpec((c,), full),
        ],
        out_specs=pl.BlockSpec((1, 1), lambda i: (0, 0)),
        out_shape=jax.ShapeDtypeStruct((1, 1), jnp.float32),
    )(pcl_prob, labels, w, pc_labels, pc_probs, pc_count, img_w, im_labels)
    return out[0, 0]


def kernel(pcl_prob, labels, cls_loss_weights, gt_assignment, pc_labels,
           pc_probs, pc_count, img_cls_loss_weights, im_labels_real):
    n, c = pcl_prob.shape
    p = pc_labels.shape[0]
    return _loss(pcl_prob, labels, cls_loss_weights, pc_labels, pc_probs,
                 pc_count, img_cls_loss_weights, im_labels_real,
                 n=n, c=c, p=p, bn=4096)


# speedup vs baseline: 4.5809x; 1.0419x over previous
import functools

import jax
import jax.numpy as jnp
from jax import lax
from jax.experimental import pallas as pl


def _body(pcl_ref, lab_ref, w_ref, pclab_ref, pcp_ref, pcc_ref, imw_ref,
          im_ref, out_ref, *, n, c, p, bn):
    i = pl.program_id(0)
    base = i * bn
    im_r = im_ref[...].reshape(1, c)
    gate = (im_r[:, 0:1] != 0.0).astype(jnp.float32)

    x = pcl_ref[:, 0:1]
    z = jnp.log(jnp.where(x > 0.0, x, 1.0))
    valid1 = base + lax.broadcasted_iota(jnp.int32, (bn,), 0) < n
    wm = jnp.where(valid1 & (lab_ref[...] == 0), w_ref[...], 0.0)
    bg_part = lax.dot_general(
        wm.reshape(1, bn), z,
        dimension_numbers=(((1,), (0,)), ((), ())),
        preferred_element_type=jnp.float32)

    @pl.when(i == 0)
    def _():
        pclab = pclab_ref[...].reshape(1, p)
        imnz = (im_r != 0.0).astype(jnp.float32)
        onehot = (lax.broadcasted_iota(jnp.int32, (c, p), 0)
                  == pclab).astype(jnp.float32)
        im_at_nz = lax.dot_general(
            imnz, onehot,
            dimension_numbers=(((1,), (0,)), ((), ())),
            preferred_element_type=jnp.float32)
        pcp = pcp_ref[...].reshape(1, p)
        fg_mask = ((im_at_nz > 0.5) & (pclab > 0)
                   & (pcc_ref[...].reshape(1, p) > 0.0))
        fg = jnp.sum(
            jnp.where(fg_mask,
                      imw_ref[...].reshape(1, p) * jnp.log(pcp), 0.0),
            keepdims=True)
        out_ref[...] = fg * jnp.float32(-1.0 / n)

    out_ref[...] += (gate * bg_part) * jnp.float32(-1.0 / n)


@functools.partial(jax.jit, static_argnames=("n", "c", "p", "bn"))
def _loss(pcl_prob, labels, w, pc_labels, pc_probs, pc_count, img_w,
          im_labels, *, n, c, p, bn):
    grid = -(-n // bn)
    full = lambda i: (0,)
    out = pl.pallas_call(
        functools.partial(_body, n=n, c=c, p=p, bn=bn),
        grid=(grid,),
        in_specs=[
            pl.BlockSpec((bn, c), lambda i: (i, 0)),
            pl.BlockSpec((bn,), lambda i: (i,)),
            pl.BlockSpec((bn,), lambda i: (i,)),
            pl.BlockSpec((p,), full),
            pl.BlockSpec((p,), full),
            pl.BlockSpec((p,), full),
            pl.BlockSpec((p,), full),
            pl.BlockSpec((c,), full),
        ],
        out_specs=pl.BlockSpec((1, 1), lambda i: (0, 0)),
        out_shape=jax.ShapeDtypeStruct((1, 1), jnp.float32),
    )(pcl_prob, labels, w, pc_labels, pc_probs, pc_count, img_w, im_labels)
    return out[0, 0]


def kernel(pcl_prob, labels, cls_loss_weights, gt_assignment, pc_labels,
           pc_probs, pc_count, img_cls_loss_weights, im_labels_real):
    n, c = pcl_prob.shape
    p = pc_labels.shape[0]
    return _loss(pcl_prob, labels, cls_loss_weights, pc_labels, pc_probs,
                 pc_count, img_cls_loss_weights, im_labels_real,
                 n=n, c=c, p=p, bn=20480)
